# baseline (speedup 1.0000x reference)
import functools

import jax
import jax.numpy as jnp
from jax import lax
from jax.experimental import pallas as pl
from jax.experimental.pallas import tpu as pltpu
from jax.experimental.pallas import tpu_sc as plsc

B, L, V, D, H, C = 4096, 50, 1000000, 64, 256, 10

CP = 16
PK = 8

NC, NS = 2, 16
NW = NC * NS
BAGS_PER_W = B // NW
BAGS_PER_STEP = 2
IDX_PER_STEP = BAGS_PER_STEP * L
OFF_W = 112
STEPS = BAGS_PER_W // BAGS_PER_STEP

_DN = lax.GatherDimensionNumbers(
    offset_dims=(), collapsed_slice_dims=(0,), start_index_map=(0,))


def _splat(vec, kvec):
    return lax.gather(vec, kvec, _DN, (1,),
                      mode=lax.GatherScatterMode.PROMISE_IN_BOUNDS)


_PRJ_BLK = 32768


_PRJ_Q = _PRJ_BLK // PK
_PRJ_GRID = (V + _PRJ_BLK - 1) // _PRJ_BLK
PKV = _PRJ_GRID * _PRJ_Q


def _prj_body(tt_ref, w1_ref, w2_ref, out_ref):
    m = lax.dot_general(w2_ref[...], w1_ref[...],
                        (((1,), (0,)), ((), ())),
                        preferred_element_type=jnp.float32)
    mp = jnp.pad(m, ((0, CP - C), (0, 0)))
    z = lax.dot_general(mp, tt_ref[...],
                        (((1,), (0,)), ((), ())),
                        preferred_element_type=jnp.float32)
    for j in range(PK):
        out_ref[:, j * CP:(j + 1) * CP] = (
            z[:, j * _PRJ_Q:(j + 1) * _PRJ_Q].T)


def _tc_project(tableT, W1, W2):
    return pl.pallas_call(
        _prj_body,
        out_shape=jax.ShapeDtypeStruct((PKV, PK * CP), jnp.float32),
        grid=(_PRJ_GRID,),
        in_specs=[
            pl.BlockSpec((D, _PRJ_BLK), lambda i: (0, i)),
            pl.BlockSpec((H, D), lambda i: (0, 0)),
            pl.BlockSpec((C, H), lambda i: (0, 0)),
        ],
        out_specs=pl.BlockSpec((_PRJ_Q, PK * CP), lambda i: (i, 0)),
    )(tableT, W1, W2)


def _sc_gather_mean(idx_q, off, pk):
    mesh = plsc.VectorSubcoreMesh(core_axis_name="c", subcore_axis_name="s")

    @functools.partial(
        pl.kernel,
        out_type=jax.ShapeDtypeStruct((B, CP), jnp.float32),
        mesh=mesh,
        scratch_types=[
            pltpu.VMEM((STEPS, IDX_PER_STEP), jnp.int32),
            pltpu.VMEM((STEPS, OFF_W), jnp.int32),
            pltpu.VMEM((IDX_PER_STEP, PK * CP), jnp.float32),
            pltpu.VMEM((IDX_PER_STEP, PK * CP), jnp.float32),
            pltpu.VMEM((BAGS_PER_W, CP), jnp.float32),
            pltpu.SemaphoreType.DMA,
            pltpu.SemaphoreType.DMA,
        ],
        compiler_params=pltpu.CompilerParams(needs_layout_passes=False),
    )
    def k(pk_hbm, x_hbm, o_hbm, out_hbm, idx_v, off_v, rows_a, rows_b,
          acc_v, sem_a, sem_b):
        wid = lax.axis_index("s") * NC + lax.axis_index("c")
        pltpu.sync_copy(x_hbm.at[pl.ds(wid * STEPS, STEPS)], idx_v)
        pltpu.sync_copy(o_hbm.at[pl.ds(wid * STEPS, STEPS)], off_v)

        def fire(g, buf, sem):
            pltpu.async_copy(pk_hbm.at[idx_v.at[g]], buf, sem)

        def wait(g, buf, sem):
            pltpu.make_async_copy(pk_hbm.at[idx_v.at[g]], buf, sem).wait()

        def accum(g, buf):
            iota16 = lax.iota(jnp.int32, 16)
            zero16 = iota16 * 0
            zcol = zero16.reshape(16, 1)
            for b in range(BAGS_PER_STEP):
                acc = jnp.zeros((16,), jnp.float32)
                ov16 = None
                last_chunk = -1
                for l in range(L):
                    r = b * L + l
                    if r // 16 != last_chunk:
                        last_chunk = r // 16
                        ov16 = off_v[g, pl.ds(16 * last_chunk, 16)]
                    off16 = _splat(ov16, zcol + (r % 16))
                    rowc = zero16 + r
                    acc = acc + plsc.load_gather(buf, [rowc, off16 + iota16])
                acc_v[g * BAGS_PER_STEP + b, :] = acc * (1.0 / L)

        fire(0, rows_a, sem_a)

        def step2(g2, carry):
            g = g2 * 2
            fire(g + 1, rows_b, sem_b)
            wait(g, rows_a, sem_a)
            accum(g, rows_a)

            @pl.when(g2 < STEPS // 2 - 1)
            def _():
                fire(g + 2, rows_a, sem_a)

            wait(g + 1, rows_b, sem_b)
            accum(g + 1, rows_b)
            return carry

        lax.fori_loop(0, STEPS // 2, step2, 0)
        pltpu.sync_copy(acc_v, out_hbm.at[pl.ds(wid * BAGS_PER_W, BAGS_PER_W)])

    return k(pk, idx_q, off)


_SM_BLK = 1024


def _sm_body(s_ref, w2_ref, b1_ref, b2_ref, out_ref):
    c = lax.dot_general(b1_ref[...], w2_ref[...],
                        (((1,), (1,)), ((), ())),
                        preferred_element_type=jnp.float32) + b2_ref[...]
    o = s_ref[...][:, :C] + c
    m = jnp.max(o, axis=-1, keepdims=True)
    e = jnp.exp(o - m)
    out_ref[...] = e / jnp.sum(e, axis=-1, keepdims=True)


def _tc_softmax(sums, W2, b1, b2):
    return pl.pallas_call(
        _sm_body,
        out_shape=jax.ShapeDtypeStruct((B, C), jnp.float32),
        grid=(B // _SM_BLK,),
        in_specs=[
            pl.BlockSpec((_SM_BLK, CP), lambda i: (i, 0)),
            pl.BlockSpec((C, H), lambda i: (0, 0)),
            pl.BlockSpec((1, H), lambda i: (0, 0)),
            pl.BlockSpec((1, C), lambda i: (0, 0)),
        ],
        out_specs=pl.BlockSpec((_SM_BLK, C), lambda i: (i, 0)),
    )(sums, W2, b1, b2)


def kernel(x, table, W1, b1, W2, b2):
    xi = x.astype(jnp.int32)
    rows = ((xi >> 15) * _PRJ_Q) | (xi & (_PRJ_Q - 1))
    lanes = ((xi >> 12) & (PK - 1)) * CP
    idx_q = rows.reshape(-1, IDX_PER_STEP)
    off = jnp.pad(lanes.reshape(-1, IDX_PER_STEP),
                  ((0, 0), (0, OFF_W - IDX_PER_STEP)))
    pk = _tc_project(table.T, W1, W2)
    sums = _sc_gather_mean(idx_q, off, pk)
    return _tc_softmax(sums, W2, b1.reshape(1, H), b2.reshape(1, C))


# speedup vs baseline: 1.0245x; 1.0245x over previous
import functools

import jax
import jax.numpy as jnp
from jax import lax
from jax.experimental import pallas as pl
from jax.experimental.pallas import tpu as pltpu
from jax.experimental.pallas import tpu_sc as plsc

B, L, V, D, H, C = 4096, 50, 1000000, 64, 256, 10

CP = 16
PK = 8

NC, NS = 2, 16
NW = NC * NS
BAGS_PER_W = B // NW
BAGS_PER_STEP = 2
IDX_PER_STEP = BAGS_PER_STEP * L
OFF_W = 112
STEPS = BAGS_PER_W // BAGS_PER_STEP

_DN = lax.GatherDimensionNumbers(
    offset_dims=(), collapsed_slice_dims=(0,), start_index_map=(0,))


def _splat(vec, kvec):
    return lax.gather(vec, kvec, _DN, (1,),
                      mode=lax.GatherScatterMode.PROMISE_IN_BOUNDS)


_PRJ_BLK = 32768


_PRJ_Q = _PRJ_BLK // PK
_PRJ_GRID = (V + _PRJ_BLK - 1) // _PRJ_BLK
PKV = _PRJ_GRID * _PRJ_Q


def _prj_body(tt_ref, w1_ref, w2_ref, out_ref):
    m = lax.dot_general(w2_ref[...], w1_ref[...],
                        (((1,), (0,)), ((), ())),
                        preferred_element_type=jnp.float32)
    mp = jnp.pad(m, ((0, CP - C), (0, 0)))
    for j in range(PK):
        zj = lax.dot_general(tt_ref[:, j * _PRJ_Q:(j + 1) * _PRJ_Q], mp,
                             (((0,), (1,)), ((), ())),
                             preferred_element_type=jnp.float32)
        out_ref[:, j * CP:(j + 1) * CP] = zj


def _tc_project(tableT, W1, W2):
    return pl.pallas_call(
        _prj_body,
        out_shape=jax.ShapeDtypeStruct((PKV, PK * CP), jnp.float32),
        grid=(_PRJ_GRID,),
        in_specs=[
            pl.BlockSpec((D, _PRJ_BLK), lambda i: (0, i)),
            pl.BlockSpec((H, D), lambda i: (0, 0)),
            pl.BlockSpec((C, H), lambda i: (0, 0)),
        ],
        out_specs=pl.BlockSpec((_PRJ_Q, PK * CP), lambda i: (i, 0)),
        compiler_params=pltpu.CompilerParams(
            fuse_transposed_lhs_in_matmul=True),
    )(tableT, W1, W2)


def _sc_gather_mean(idx_q, off, pk):
    mesh = plsc.VectorSubcoreMesh(core_axis_name="c", subcore_axis_name="s")

    @functools.partial(
        pl.kernel,
        out_type=jax.ShapeDtypeStruct((B, CP), jnp.float32),
        mesh=mesh,
        scratch_types=[
            pltpu.VMEM((STEPS, IDX_PER_STEP), jnp.int32),
            pltpu.VMEM((STEPS, OFF_W), jnp.int32),
            pltpu.VMEM((IDX_PER_STEP, PK * CP), jnp.float32),
            pltpu.VMEM((IDX_PER_STEP, PK * CP), jnp.float32),
            pltpu.VMEM((BAGS_PER_W, CP), jnp.float32),
            pltpu.SemaphoreType.DMA,
            pltpu.SemaphoreType.DMA,
        ],
        compiler_params=pltpu.CompilerParams(needs_layout_passes=False),
    )
    def k(pk_hbm, x_hbm, o_hbm, out_hbm, idx_v, off_v, rows_a, rows_b,
          acc_v, sem_a, sem_b):
        wid = lax.axis_index("s") * NC + lax.axis_index("c")
        pltpu.sync_copy(x_hbm.at[pl.ds(wid * STEPS, STEPS)], idx_v)
        pltpu.sync_copy(o_hbm.at[pl.ds(wid * STEPS, STEPS)], off_v)

        def fire(g, buf, sem):
            pltpu.async_copy(pk_hbm.at[idx_v.at[g]], buf, sem)

        def wait(g, buf, sem):
            pltpu.make_async_copy(pk_hbm.at[idx_v.at[g]], buf, sem).wait()

        def accum(g, buf):
            iota16 = lax.iota(jnp.int32, 16)
            zero16 = iota16 * 0
            zcol = zero16.reshape(16, 1)
            for b in range(BAGS_PER_STEP):
                acc = jnp.zeros((16,), jnp.float32)
                ov16 = None
                last_chunk = -1
                for l in range(L):
                    r = b * L + l
                    if r // 16 != last_chunk:
                        last_chunk = r // 16
                        ov16 = off_v[g, pl.ds(16 * last_chunk, 16)]
                    off16 = _splat(ov16, zcol + (r % 16))
                    rowc = zero16 + r
                    acc = acc + plsc.load_gather(buf, [rowc, off16 + iota16])
                acc_v[g * BAGS_PER_STEP + b, :] = acc * (1.0 / L)

        fire(0, rows_a, sem_a)

        def step2(g2, carry):
            g = g2 * 2
            fire(g + 1, rows_b, sem_b)
            wait(g, rows_a, sem_a)
            accum(g, rows_a)

            @pl.when(g2 < STEPS // 2 - 1)
            def _():
                fire(g + 2, rows_a, sem_a)

            wait(g + 1, rows_b, sem_b)
            accum(g + 1, rows_b)
            return carry

        lax.fori_loop(0, STEPS // 2, step2, 0)
        pltpu.sync_copy(acc_v, out_hbm.at[pl.ds(wid * BAGS_PER_W, BAGS_PER_W)])

    return k(pk, idx_q, off)


_SM_BLK = 1024


def _sm_body(s_ref, w2_ref, b1_ref, b2_ref, out_ref):
    c = lax.dot_general(b1_ref[...], w2_ref[...],
                        (((1,), (1,)), ((), ())),
                        preferred_element_type=jnp.float32) + b2_ref[...]
    o = s_ref[...][:, :C] + c
    m = jnp.max(o, axis=-1, keepdims=True)
    e = jnp.exp(o - m)
    out_ref[...] = e / jnp.sum(e, axis=-1, keepdims=True)


def _tc_softmax(sums, W2, b1, b2):
    return pl.pallas_call(
        _sm_body,
        out_shape=jax.ShapeDtypeStruct((B, C), jnp.float32),
        grid=(B // _SM_BLK,),
        in_specs=[
            pl.BlockSpec((_SM_BLK, CP), lambda i: (i, 0)),
            pl.BlockSpec((C, H), lambda i: (0, 0)),
            pl.BlockSpec((1, H), lambda i: (0, 0)),
            pl.BlockSpec((1, C), lambda i: (0, 0)),
        ],
        out_specs=pl.BlockSpec((_SM_BLK, C), lambda i: (i, 0)),
    )(sums, W2, b1, b2)


def kernel(x, table, W1, b1, W2, b2):
    xi = x.astype(jnp.int32)
    rows = ((xi >> 15) * _PRJ_Q) | (xi & (_PRJ_Q - 1))
    lanes = ((xi >> 12) & (PK - 1)) * CP
    idx_q = rows.reshape(-1, IDX_PER_STEP)
    off = jnp.pad(lanes.reshape(-1, IDX_PER_STEP),
                  ((0, 0), (0, OFF_W - IDX_PER_STEP)))
    pk = _tc_project(table.T, W1, W2)
    sums = _sc_gather_mean(idx_q, off, pk)
    return _tc_softmax(sums, W2, b1.reshape(1, H), b2.reshape(1, C))


# speedup vs baseline: 2.0658x; 2.0164x over previous
import functools

import jax
import jax.numpy as jnp
from jax import lax
from jax.experimental import pallas as pl
from jax.experimental.pallas import tpu as pltpu
from jax.experimental.pallas import tpu_sc as plsc

B, L, V, D, H, C = 4096, 50, 1000000, 64, 256, 10

CP = 16
PK = 8

NC, NS = 2, 16
NW = NC * NS
BAGS_PER_W = B // NW
BAGS_PER_STEP = 2
IDX_PER_STEP = BAGS_PER_STEP * L
STEPS = BAGS_PER_W // BAGS_PER_STEP

_PRJ_BLK = 32768


_PRJ_Q = _PRJ_BLK // PK
_PRJ_GRID = (V + _PRJ_BLK - 1) // _PRJ_BLK
PKV = _PRJ_GRID * _PRJ_Q


def _prj_body(*refs):
    tt_refs, w1_ref, w2_ref, out_ref = refs[:PK], refs[PK], refs[PK + 1], \
        refs[PK + 2]
    m = lax.dot_general(w2_ref[...], w1_ref[...],
                        (((1,), (0,)), ((), ())),
                        preferred_element_type=jnp.float32)
    mt = jnp.pad(m, ((0, CP - C), (0, 0))).T
    lhs = jnp.concatenate([r[...] for r in tt_refs], axis=0)
    rblk = jnp.concatenate(
        [jnp.pad(mt, ((j * D, (PK - 1 - j) * D), (0, 0)))
         for j in range(PK)], axis=1)
    out_ref[...] = lax.dot_general(lhs, rblk,
                                   (((0,), (0,)), ((), ())),
                                   preferred_element_type=jnp.float32)


def _tc_project(tableT, W1, W2):
    last_blk = (V + _PRJ_Q - 1) // _PRJ_Q - 1

    def make_spec(j):
        return pl.BlockSpec(
            (D, _PRJ_Q),
            lambda i, j=j: (0, jnp.minimum(PK * i + j, last_blk)))

    return pl.pallas_call(
        _prj_body,
        out_shape=jax.ShapeDtypeStruct((PKV, PK * CP), jnp.float32),
        grid=(_PRJ_GRID,),
        in_specs=[make_spec(j) for j in range(PK)] + [
            pl.BlockSpec((H, D), lambda i: (0, 0)),
            pl.BlockSpec((C, H), lambda i: (0, 0)),
        ],
        out_specs=pl.BlockSpec((_PRJ_Q, PK * CP), lambda i: (i, 0)),
        compiler_params=pltpu.CompilerParams(
            fuse_transposed_lhs_in_matmul=True),
    )(*([tableT] * PK), W1, W2)


def _sc_gather_mean(idx_q, pk2):
    mesh = plsc.VectorSubcoreMesh(core_axis_name="c", subcore_axis_name="s")

    @functools.partial(
        pl.kernel,
        out_type=jax.ShapeDtypeStruct((B, CP), jnp.float32),
        mesh=mesh,
        scratch_types=[
            pltpu.VMEM((STEPS, IDX_PER_STEP), jnp.int32),
            pltpu.VMEM((IDX_PER_STEP, CP), jnp.float32),
            pltpu.VMEM((IDX_PER_STEP, CP), jnp.float32),
            pltpu.VMEM((BAGS_PER_W, CP), jnp.float32),
            pltpu.SemaphoreType.DMA,
            pltpu.SemaphoreType.DMA,
        ],
        compiler_params=pltpu.CompilerParams(
            needs_layout_passes=False, use_tc_tiling_on_sc=False),
    )
    def k(pk_hbm, x_hbm, out_hbm, idx_v, rows_a, rows_b,
          acc_v, sem_a, sem_b):
        wid = lax.axis_index("s") * NC + lax.axis_index("c")
        pltpu.sync_copy(x_hbm.at[pl.ds(wid * STEPS, STEPS)], idx_v)

        def fire(g, buf, sem):
            pltpu.async_copy(pk_hbm.at[idx_v.at[g]], buf, sem)

        def wait(g, buf, sem):
            pltpu.make_async_copy(pk_hbm.at[idx_v.at[g]], buf, sem).wait()

        def accum(g, buf):
            for b in range(BAGS_PER_STEP):
                acc = jnp.zeros((16,), jnp.float32)
                for l in range(L):
                    acc = acc + buf[b * L + l, :]
                acc_v[g * BAGS_PER_STEP + b, :] = acc * (1.0 / L)

        fire(0, rows_a, sem_a)

        def step2(g2, carry):
            g = g2 * 2
            fire(g + 1, rows_b, sem_b)
            wait(g, rows_a, sem_a)
            accum(g, rows_a)

            @pl.when(g2 < STEPS // 2 - 1)
            def _():
                fire(g + 2, rows_a, sem_a)

            wait(g + 1, rows_b, sem_b)
            accum(g + 1, rows_b)
            return carry

        lax.fori_loop(0, STEPS // 2, step2, 0)
        pltpu.sync_copy(acc_v, out_hbm.at[pl.ds(wid * BAGS_PER_W, BAGS_PER_W)])

    return k(pk2, idx_q)


_SM_BLK = 1024


def _sm_body(s_ref, w2_ref, b1_ref, b2_ref, out_ref):
    c = lax.dot_general(b1_ref[...], w2_ref[...],
                        (((1,), (1,)), ((), ())),
                        preferred_element_type=jnp.float32) + b2_ref[...]
    o = s_ref[...][:, :C] + c
    m = jnp.max(o, axis=-1, keepdims=True)
    e = jnp.exp(o - m)
    out_ref[...] = e / jnp.sum(e, axis=-1, keepdims=True)


def _tc_softmax(sums, W2, b1, b2):
    return pl.pallas_call(
        _sm_body,
        out_shape=jax.ShapeDtypeStruct((B, C), jnp.float32),
        grid=(B // _SM_BLK,),
        in_specs=[
            pl.BlockSpec((_SM_BLK, CP), lambda i: (i, 0)),
            pl.BlockSpec((C, H), lambda i: (0, 0)),
            pl.BlockSpec((1, H), lambda i: (0, 0)),
            pl.BlockSpec((1, C), lambda i: (0, 0)),
        ],
        out_specs=pl.BlockSpec((_SM_BLK, C), lambda i: (i, 0)),
    )(sums, W2, b1, b2)


def kernel(x, table, W1, b1, W2, b2):
    xi = x.astype(jnp.int32)
    rows = ((xi >> 15) * _PRJ_Q) | (xi & (_PRJ_Q - 1))
    grp = (xi >> 12) & (PK - 1)
    idx_q = ((rows * PK) | grp).reshape(-1, IDX_PER_STEP)
    pk = _tc_project(table.T, W1, W2)
    sums = _sc_gather_mean(idx_q, pk.reshape(PKV * PK, CP))
    return _tc_softmax(sums, W2, b1.reshape(1, H), b2.reshape(1, C))
